# baseline (speedup 1.0000x reference)
import functools

import jax
import jax.numpy as jnp
from jax import lax
from jax.experimental import pallas as pl
from jax.experimental.pallas import tpu as pltpu
from jax.experimental.pallas import tpu_sc as plsc


def _make_sc_kernel(b: int, h: int, d: int, v4: int, hc: int):
    info = plsc.get_sparse_core_info()
    nc, ns = info.num_cores, info.num_subcores
    nw = nc * ns
    assert b % (nw * 16) == 0
    bw = b // nw
    nbq = bw // 16
    assert h % hc == 0
    n_chunks = h // hc
    assert d == 32
    mesh = plsc.VectorSubcoreMesh(core_axis_name="c", subcore_axis_name="s")

    @functools.partial(
        pl.kernel,
        mesh=mesh,
        out_type=jax.ShapeDtypeStruct((h, d, b), jnp.float32),
        compiler_params=pltpu.CompilerParams(use_tc_tiling_on_sc=True,
                                             needs_layout_passes=False),
        scratch_types=(
            [pltpu.VMEM((hc, bw), jnp.int32),
             pltpu.VMEM((hc, bw), jnp.int32),
             pltpu.VMEM((hc, bw), jnp.float32),
             pltpu.VMEM((hc * bw, 4 * d), jnp.float32),
             pltpu.VMEM((hc, d, bw), jnp.float32)]
            + [pltpu.SemaphoreType.DMA] * hc
            + [pltpu.SemaphoreType.DMA]
        ),
    )
    def sc_kernel(idx_hbm, score_hbm, table_hbm, out_hbm,
                  idx_v, idx4_v, score_v, rows_v, out_v, *sems):
        row_sems, out_sem = sems[:hc], sems[hc]
        wid = lax.axis_index("s") * nc + lax.axis_index("c")
        b0 = wid * bw
        iota = lax.iota(jnp.int32, 16)

        def chunk_body(g, carry):
            h0 = g * hc
            pltpu.sync_copy(idx_hbm.at[pl.ds(h0, hc), pl.ds(b0, bw)], idx_v)
            pltpu.sync_copy(score_hbm.at[pl.ds(h0, hc), pl.ds(b0, bw)],
                            score_v)
            for r in range(hc):
                for bq in range(nbq):
                    bo = bq * 16
                    idx4_v[r, pl.ds(bo, 16)] = jnp.right_shift(
                        idx_v[r, pl.ds(bo, 16)], 2)
            descs = []
            for r in range(hc):
                descs.append(pltpu.async_copy(
                    table_hbm.at[idx4_v.at[r]],
                    rows_v.at[pl.ds(r * bw, bw)], row_sems[r]))

            @pl.when(g > 0)
            def _():
                pltpu.make_async_copy(
                    out_v, out_hbm.at[pl.ds(0, hc), :, pl.ds(b0, bw)],
                    out_sem).wait()

            for r in range(hc):
                descs[r].wait()
                for bq in range(nbq):
                    bo = bq * 16
                    idxvec = idx_v[r, pl.ds(bo, 16)]
                    svec = score_v[r, pl.ds(bo, 16)]
                    rvec = iota + (r * bw + bo)
                    cvec = jnp.bitwise_and(idxvec, 3) * d
                    for e in range(d):
                        vals = plsc.load_gather(rows_v, [rvec, cvec])
                        out_v[r, e, pl.ds(bo, 16)] = vals * svec
                        if e != d - 1:
                            cvec = cvec + 1
            pltpu.async_copy(
                out_v, out_hbm.at[pl.ds(h0, hc), :, pl.ds(b0, bw)], out_sem)
            return carry

        lax.fori_loop(0, n_chunks, chunk_body, 0)
        pltpu.make_async_copy(
            out_v, out_hbm.at[pl.ds(0, hc), :, pl.ds(b0, bw)],
            out_sem).wait()

    return sc_kernel


def kernel(v_e, v_score, table):
    b, h = v_e.shape
    v, d = table.shape
    idx_t = v_e.T.astype(jnp.int32)
    score_t = v_score.T.astype(jnp.float32)
    table4 = table.reshape(v // 4, 4 * d)
    out_t = _make_sc_kernel(b, h, d, v // 4, hc=4)(idx_t, score_t, table4)
    return jnp.transpose(out_t, (2, 0, 1))


# speedup vs baseline: 1.0122x; 1.0122x over previous
import functools

import jax
import jax.numpy as jnp
from jax import lax
from jax.experimental import pallas as pl
from jax.experimental.pallas import tpu as pltpu
from jax.experimental.pallas import tpu_sc as plsc


def _make_sc_kernel(b: int, h: int, d: int, hc: int):
    info = plsc.get_sparse_core_info()
    nc, ns = info.num_cores, info.num_subcores
    nw = nc * ns
    assert b % (nw * 16) == 0
    bw = b // nw
    nbq = bw // 16
    assert h % hc == 0
    n_chunks = h // hc
    mesh = plsc.VectorSubcoreMesh(core_axis_name="c", subcore_axis_name="s")

    @functools.partial(
        pl.kernel,
        mesh=mesh,
        out_type=jax.ShapeDtypeStruct((h, d, b), jnp.float32),
        compiler_params=pltpu.CompilerParams(needs_layout_passes=False),
        scratch_types=(
            [pltpu.VMEM((hc, bw), jnp.int32),
             pltpu.VMEM((hc, bw), jnp.int32),
             pltpu.VMEM((hc, bw), jnp.float32),
             pltpu.VMEM((hc * bw, 4 * d), jnp.float32),
             pltpu.VMEM((hc, d, bw), jnp.float32)]
            + [pltpu.SemaphoreType.DMA] * hc
            + [pltpu.SemaphoreType.DMA]
        ),
    )
    def sc_kernel(slice_hbm, col_hbm, score_hbm, table_hbm, out_hbm,
                  slice_v, col_v, score_v, rows_v, out_v, *sems):
        row_sems, out_sem = sems[:hc], sems[hc]
        wid = lax.axis_index("s") * nc + lax.axis_index("c")
        b0 = wid * bw
        iota = lax.iota(jnp.int32, 16)

        def chunk_body(g, carry):
            h0 = g * hc
            pltpu.sync_copy(slice_hbm.at[pl.ds(h0, hc), pl.ds(b0, bw)],
                            slice_v)
            pltpu.sync_copy(col_hbm.at[pl.ds(h0, hc), pl.ds(b0, bw)], col_v)
            pltpu.sync_copy(score_hbm.at[pl.ds(h0, hc), pl.ds(b0, bw)],
                            score_v)
            descs = []
            for r in range(hc):
                descs.append(pltpu.async_copy(
                    table_hbm.at[slice_v.at[r]],
                    rows_v.at[pl.ds(r * bw, bw)], row_sems[r]))

            @pl.when(g > 0)
            def _():
                pltpu.make_async_copy(
                    out_v, out_hbm.at[pl.ds(0, hc), :, pl.ds(b0, bw)],
                    out_sem).wait()

            for r in range(hc):
                descs[r].wait()

                def grp_body(bq, c):
                    bo = bq * 16
                    svec = score_v[r, pl.ds(bo, 16)]
                    rvec = iota + (r * bw + bo)
                    cvec = col_v[r, pl.ds(bo, 16)]
                    for e in range(d):
                        vals = plsc.load_gather(rows_v, [rvec, cvec])
                        out_v[r, e, pl.ds(bo, 16)] = vals * svec
                        if e != d - 1:
                            cvec = cvec + 1
                    return c

                lax.fori_loop(0, nbq, grp_body, 0)
            pltpu.async_copy(
                out_v, out_hbm.at[pl.ds(h0, hc), :, pl.ds(b0, bw)], out_sem)
            return carry

        lax.fori_loop(0, n_chunks, chunk_body, 0)
        pltpu.make_async_copy(
            out_v, out_hbm.at[pl.ds(0, hc), :, pl.ds(b0, bw)],
            out_sem).wait()

    return sc_kernel


def kernel(v_e, v_score, table):
    b, h = v_e.shape
    v, d = table.shape
    idx_t = v_e.T.astype(jnp.int32)
    slice_t = idx_t >> 2
    col_t = (idx_t & 3) * d
    score_t = v_score.T.astype(jnp.float32)
    table4 = table.reshape(v // 4, 4 * d)
    out_t = _make_sc_kernel(b, h, d, hc=4)(slice_t, col_t, score_t, table4)
    return jnp.transpose(out_t, (2, 0, 1))
